# baseline (speedup 1.0000x reference)
import jax
import jax.numpy as jnp
from jax.experimental import pallas as pl
from jax.experimental.pallas import tpu as pltpu

_LOG2E = 1.4426950408889634


def _dot_tl(aT, b):
    return jax.lax.dot_general(
        aT, b, (((0,), (0,)), ((), ())), preferred_element_type=jnp.float32
    )


def _varifold_batch(x1t_ref, n1t_ref, x2t_ref, n2t_ref, out_ref):
    x1t = x1t_ref[0]
    n1t = n1t_ref[0]
    x2 = x2t_ref[0]
    n2 = n2t_ref[0]

    T = x1t.shape[1]
    N2 = x2.shape[1]
    x1sqT = jnp.sum(x1t * x1t, axis=0, keepdims=True)
    x2sq = jnp.sum(x2 * x2, axis=0, keepdims=True)

    lhsT = jnp.concatenate(
        [x1t, jnp.ones((1, T), dtype=jnp.float32), x1sqT], axis=0
    )
    rhs = jnp.concatenate(
        [(2.0 * _LOG2E) * x2, -_LOG2E * x2sq,
         jnp.full((1, N2), -_LOG2E, dtype=jnp.float32)],
        axis=0,
    )

    arg = _dot_tl(lhsT, rhs)
    dotn = _dot_tl(n1t, n2)

    s = jnp.exp2(arg) * (dotn * dotn)

    r = s.shape[0]
    while r > 8:
        r //= 2
        s = s[:r, :] + s[r:, :]
    c = s.shape[1]
    while c > 128:
        c //= 2
        s = s[:, :c] + s[:, c:]

    out_ref[0] = jnp.sum(s, keepdims=True)


def kernel(xyz1, xyz2, nor1, nor2):
    B, N1, _ = xyz1.shape
    N2 = xyz2.shape[1]

    x1t = jnp.swapaxes(xyz1, 1, 2)
    n1t = jnp.swapaxes(nor1, 1, 2)
    x2t = jnp.swapaxes(xyz2, 1, 2)
    n2t = jnp.swapaxes(nor2, 1, 2)

    out = pl.pallas_call(
        _varifold_batch,
        grid=(B,),
        in_specs=[
            pl.BlockSpec((1, 3, N1), lambda b: (b, 0, 0)),
            pl.BlockSpec((1, 3, N1), lambda b: (b, 0, 0)),
            pl.BlockSpec((1, 3, N2), lambda b: (b, 0, 0)),
            pl.BlockSpec((1, 3, N2), lambda b: (b, 0, 0)),
        ],
        out_specs=pl.BlockSpec((1, 1, 1), lambda b: (b, 0, 0)),
        out_shape=jax.ShapeDtypeStruct((B, 1, 1), jnp.float32),
        compiler_params=pltpu.CompilerParams(dimension_semantics=("parallel",)),
    )(x1t, n1t, x2t, n2t)
    return out[:, 0, 0]


# speedup vs baseline: 1.0033x; 1.0033x over previous
import jax
import jax.numpy as jnp
from jax.experimental import pallas as pl

_LOG2E = 1.4426950408889634


def _dot_tl(aT, b):
    return jax.lax.dot_general(
        aT, b, (((0,), (0,)), ((), ())), preferred_element_type=jnp.float32
    )


def _varifold_batch(x1t_ref, n1t_ref, x2t_ref, n2t_ref, out_ref):
    x1t = x1t_ref[0]
    n1t = n1t_ref[0]
    x2 = x2t_ref[0]
    n2 = n2t_ref[0]

    T = x1t.shape[1]
    N2 = x2.shape[1]
    x1sqT = jnp.sum(x1t * x1t, axis=0, keepdims=True)
    x2sq = jnp.sum(x2 * x2, axis=0, keepdims=True)

    lhsT = jnp.concatenate(
        [x1t, jnp.ones((1, T), dtype=jnp.float32), x1sqT], axis=0
    )
    rhs = jnp.concatenate(
        [(2.0 * _LOG2E) * x2, -_LOG2E * x2sq,
         jnp.full((1, N2), -_LOG2E, dtype=jnp.float32)],
        axis=0,
    )

    arg = _dot_tl(lhsT, rhs)
    dotn = _dot_tl(n1t, n2)

    s = jnp.exp2(arg) * (dotn * dotn)

    r = s.shape[0]
    while r > 8:
        r //= 2
        s = s[:r, :] + s[r:, :]
    c = s.shape[1]
    while c > 128:
        c //= 2
        s = s[:, :c] + s[:, c:]

    out_ref[0] = jnp.sum(s, keepdims=True)


def kernel(xyz1, xyz2, nor1, nor2):
    B, N1, _ = xyz1.shape
    N2 = xyz2.shape[1]

    x1t = jnp.swapaxes(xyz1, 1, 2)
    n1t = jnp.swapaxes(nor1, 1, 2)
    x2t = jnp.swapaxes(xyz2, 1, 2)
    n2t = jnp.swapaxes(nor2, 1, 2)

    out = pl.pallas_call(
        _varifold_batch,
        grid=(B,),
        in_specs=[
            pl.BlockSpec((1, 3, N1), lambda b: (b, 0, 0)),
            pl.BlockSpec((1, 3, N1), lambda b: (b, 0, 0)),
            pl.BlockSpec((1, 3, N2), lambda b: (b, 0, 0)),
            pl.BlockSpec((1, 3, N2), lambda b: (b, 0, 0)),
        ],
        out_specs=pl.BlockSpec((1, 1, 1), lambda b: (b, 0, 0)),
        out_shape=jax.ShapeDtypeStruct((B, 1, 1), jnp.float32),
    )(x1t, n1t, x2t, n2t)
    return out[:, 0, 0]
